# baseline (speedup 1.0000x reference)
import functools

import jax
import jax.numpy as jnp
from jax import lax
from jax.experimental import pallas as pl
from jax.experimental.pallas import tpu as pltpu
from jax.experimental.pallas import tpu_sc as plsc

N_NODES = 10000
N_EDGES = 320000
D = 128

NC = 2
NS = 16
NW = NC * NS
SLABS = 2
E_SLAB = N_EDGES // SLABS
CHUNK = 40
N_PAD = 10240
ROWS_PER_TILE = N_PAD // NS
ZCHUNK = 64

BE = 3200
BN = 2000


def _mm_bias_body(a_ref, w_ref, b_ref, o_ref):
    o_ref[...] = (
        jnp.dot(a_ref[...], w_ref[...], preferred_element_type=jnp.float32)
        + b_ref[...]
    )


def _edge_proj(edge_attr, lin_Wt, lin_b):
    e = edge_attr.shape[0]
    return pl.pallas_call(
        _mm_bias_body,
        grid=(e // BE,),
        in_specs=[
            pl.BlockSpec((BE, D), lambda i: (i, 0)),
            pl.BlockSpec((D, D), lambda i: (0, 0)),
            pl.BlockSpec((1, D), lambda i: (0, 0)),
        ],
        out_specs=pl.BlockSpec((BE, D), lambda i: (i, 0)),
        out_shape=jax.ShapeDtypeStruct((e, D), jnp.float32),
    )(edge_attr, lin_Wt, lin_b.reshape(1, D))


def _combine_body(x_ref, pa0_ref, pa1_ref, pb0_ref, pb1_ref,
                  w_ref, b_ref, o_ref):
    h = (x_ref[...] + (pa0_ref[...] + pa1_ref[...])
         + (pb0_ref[...] + pb1_ref[...]))
    o_ref[...] = (
        jnp.dot(h, w_ref[...], preferred_element_type=jnp.float32) + b_ref[...]
    )


def _combine(x, parts, nn_Wt, nn_b):
    node_spec = pl.BlockSpec((BN, D), lambda i: (i, 0))
    return pl.pallas_call(
        _combine_body,
        grid=(N_NODES // BN,),
        in_specs=[node_spec] * 5 + [
            pl.BlockSpec((D, D), lambda i: (0, 0)),
            pl.BlockSpec((1, D), lambda i: (0, 0)),
        ],
        out_specs=node_spec,
        out_shape=jax.ShapeDtypeStruct((N_NODES, D), jnp.float32),
    )(x, *parts, nn_Wt, nn_b.reshape(1, D))


def _run_if(cond):
    def deco(f):
        if cond:
            f()
    return deco


_sc_mesh = plsc.VectorSubcoreMesh(core_axis_name="c", subcore_axis_name="s")


def _make_sc_aggregate(e_per_w):
    n_chunks = e_per_w // CHUNK

    @functools.partial(
        pl.kernel,
        out_type=jax.ShapeDtypeStruct((NC * N_PAD, D), jnp.float32),
        mesh=_sc_mesh,
        scratch_types=[pltpu.VMEM((CHUNK,), jnp.int32)] * 8
        + [pltpu.VMEM((CHUNK, D), jnp.float32)] * 4
        + [
            pltpu.VMEM((ZCHUNK, D), jnp.float32),
            pltpu.VMEM_SHARED((N_PAD, D), jnp.float32),
        ] + [pltpu.SemaphoreType.DMA] * 12,
    )
    def _sc_aggregate(src_hbm, dst_hbm, x_hbm, eproj_hbm, out_hbm,
                      s0, s1, s2, s3, d0, d1, d2, d3,
                      er0, er1, er2, er3, zbuf_v, acc_sh,
                      *sems):
        cid = lax.axis_index("c")
        sid = lax.axis_index("s")
        wid = sid * NC + cid
        srcb = (s0, s1, s2, s3)
        dstb = (d0, d1, d2, d3)
        erb = (er0, er1, er2, er3)
        isems = sems[0:4]
        epsems = sems[4:8]
        gasems = sems[8:12]

        zero = jnp.zeros((16,), jnp.float32)

        def zrow(r, carry):
            for g in range(D // 16):
                zbuf_v[r, pl.ds(g * 16, 16)] = zero
            return carry

        lax.fori_loop(0, ZCHUNK, zrow, 0)
        for z in range(ROWS_PER_TILE // ZCHUNK):
            ro = sid * ROWS_PER_TILE + z * ZCHUNK
            pltpu.sync_copy(zbuf_v, acc_sh.at[pl.ds(ro, ZCHUNK)])
        plsc.subcore_barrier()

        def ebase(c):
            return pl.multiple_of(wid * e_per_w + c * CHUNK, 8)

        def issue_idx(c, q):
            pltpu.async_copy(src_hbm.at[pl.ds(ebase(c), CHUNK)], srcb[q],
                             isems[q])
            pltpu.async_copy(dst_hbm.at[pl.ds(ebase(c), CHUNK)], dstb[q],
                             isems[q])

        def wait_idx(c, q):
            pltpu.make_async_copy(src_hbm.at[pl.ds(ebase(c), CHUNK)],
                                  srcb[q], isems[q]).wait()
            pltpu.make_async_copy(dst_hbm.at[pl.ds(ebase(c), CHUNK)],
                                  dstb[q], isems[q]).wait()

        def issue_eproj(c, q):
            pltpu.async_copy(eproj_hbm.at[pl.ds(ebase(c), CHUNK)], erb[q],
                             epsems[q])

        def issue_ga(c, q):
            pltpu.make_async_copy(eproj_hbm.at[pl.ds(ebase(c), CHUNK)],
                                  erb[q], epsems[q]).wait()
            wait_idx(c, q)
            pltpu.async_copy(x_hbm.at[srcb[q]], erb[q], gasems[q], add=True)

        def process(c, p, traced):
            def when(cond):
                if traced:
                    return pl.when(cond)
                return _run_if(bool(cond))

            @when(c + 2 < n_chunks)
            def _():
                issue_idx(c + 2, (p + 2) % 4)
                issue_eproj(c + 2, (p + 2) % 4)

            @when(c + 1 < n_chunks)
            def _():
                issue_ga(c + 1, (p + 1) % 4)

            pltpu.make_async_copy(x_hbm.at[srcb[p]], erb[p],
                                  gasems[p]).wait()
            er = erb[p]

            @plsc.parallel_loop(0, CHUNK, step=1)
            def row_body(r):
                for g in range(D // 16):
                    sl = pl.ds(g * 16, 16)
                    er[r, sl] = jnp.maximum(er[r, sl], 0.0)

            pltpu.sync_copy(er, acc_sh.at[dstb[p]], add=True)

        issue_idx(0, 0)
        issue_idx(1, 1)
        issue_eproj(0, 0)
        issue_eproj(1, 1)
        issue_ga(0, 0)

        def quad_body(k, carry):
            for b4 in range(4):
                process(4 * k + b4, b4, traced=True)
            return carry

        lax.fori_loop(0, n_chunks // 4, quad_body, 0)
        for c in range(4 * (n_chunks // 4), n_chunks):
            process(c, c % 4, traced=False)
        plsc.subcore_barrier()

        for z in range(ROWS_PER_TILE // ZCHUNK):
            ro = sid * ROWS_PER_TILE + z * ZCHUNK
            pltpu.sync_copy(acc_sh.at[pl.ds(ro, ZCHUNK)], zbuf_v)
            pltpu.sync_copy(
                zbuf_v, out_hbm.at[pl.ds(cid * N_PAD + ro, ZCHUNK)])

    return _sc_aggregate


_sc_slab = _make_sc_aggregate(E_SLAB // NW)


def kernel(x, edge_index, edge_attr, lin_W, lin_b, nn_W, nn_b):
    src = edge_index[0].astype(jnp.int32)
    dst = edge_index[1].astype(jnp.int32)
    lin_Wt = lin_W.T
    parts = []
    for s in range(SLABS):
        lo, hi = s * E_SLAB, (s + 1) * E_SLAB
        e_proj = _edge_proj(edge_attr[lo:hi], lin_Wt, lin_b)
        p = _sc_slab(src[lo:hi], dst[lo:hi], x, e_proj)
        parts.append(p[:N_NODES])
        parts.append(p[N_PAD:N_PAD + N_NODES])
    return _combine(x, parts, nn_W.T, nn_b)


# speedup vs baseline: 1.2086x; 1.2086x over previous
import functools

import jax
import jax.numpy as jnp
from jax import lax
from jax.experimental import pallas as pl
from jax.experimental.pallas import tpu as pltpu
from jax.experimental.pallas import tpu_sc as plsc

N_NODES = 10000
N_EDGES = 320000
D = 128

NC = 2
NS = 16
NW = NC * NS
SLABS = 1
E_SLAB = N_EDGES // SLABS
CHUNK = 40
N_PAD = 10240
ROWS_PER_TILE = N_PAD // NS
ZCHUNK = 64

BE = 2560
BN = 2000


def _mm_bias_body(a_ref, w_ref, b_ref, o_ref):
    o_ref[...] = (
        jnp.dot(a_ref[...].astype(jnp.bfloat16), w_ref[...],
                preferred_element_type=jnp.float32)
        + b_ref[...]
    )


def _edge_proj(edge_attr, lin_Wt, lin_b):
    e = edge_attr.shape[0]
    return pl.pallas_call(
        _mm_bias_body,
        grid=(e // BE,),
        in_specs=[
            pl.BlockSpec((BE, D), lambda i: (i, 0)),
            pl.BlockSpec((D, D), lambda i: (0, 0)),
            pl.BlockSpec((1, D), lambda i: (0, 0)),
        ],
        out_specs=pl.BlockSpec((BE, D), lambda i: (i, 0)),
        out_shape=jax.ShapeDtypeStruct((e, D), jnp.float32),
    )(edge_attr, lin_Wt.astype(jnp.bfloat16), lin_b.reshape(1, D))


def _combine(x, parts, nn_Wt, nn_b):
    n = len(parts)

    def body(*refs):
        x_ref, prefs, (w_ref, b_ref, o_ref) = refs[0], refs[1:1 + n], refs[1 + n:]
        h = x_ref[...]
        for p_ref in prefs:
            h = h + p_ref[...]
        o_ref[...] = (
            jnp.dot(h, w_ref[...], preferred_element_type=jnp.float32)
            + b_ref[...]
        )

    node_spec = pl.BlockSpec((BN, D), lambda i: (i, 0))
    return pl.pallas_call(
        body,
        grid=(N_NODES // BN,),
        in_specs=[node_spec] * (1 + n) + [
            pl.BlockSpec((D, D), lambda i: (0, 0)),
            pl.BlockSpec((1, D), lambda i: (0, 0)),
        ],
        out_specs=node_spec,
        out_shape=jax.ShapeDtypeStruct((N_NODES, D), jnp.float32),
    )(x, *parts, nn_Wt, nn_b.reshape(1, D))


def _run_if(cond):
    def deco(f):
        if cond:
            f()
    return deco


_sc_mesh = plsc.VectorSubcoreMesh(core_axis_name="c", subcore_axis_name="s")


def _make_sc_aggregate(e_per_w):
    n_chunks = e_per_w // CHUNK

    @functools.partial(
        pl.kernel,
        out_type=jax.ShapeDtypeStruct((NC * N_PAD, D), jnp.float32),
        mesh=_sc_mesh,
        scratch_types=[pltpu.VMEM((CHUNK,), jnp.int32)] * 8
        + [pltpu.VMEM((CHUNK, D), jnp.float32)] * 4
        + [
            pltpu.VMEM((ZCHUNK, D), jnp.float32),
            pltpu.VMEM_SHARED((N_PAD, D), jnp.float32),
        ] + [pltpu.SemaphoreType.DMA] * 13,
    )
    def _sc_aggregate(src_hbm, dst_hbm, x_hbm, eproj_hbm, out_hbm,
                      s0, s1, s2, s3, d0, d1, d2, d3,
                      er0, er1, er2, er3, zbuf_v, acc_sh,
                      *sems):
        cid = lax.axis_index("c")
        sid = lax.axis_index("s")
        wid = sid * NC + cid
        srcb = (s0, s1, s2, s3)
        dstb = (d0, d1, d2, d3)
        erb = (er0, er1, er2, er3)
        isems = sems[0:4]
        epsems = sems[4:8]
        gasems = sems[8:12]
        scsem = sems[12]

        zero = jnp.zeros((16,), jnp.float32)

        def zrow(r, carry):
            for g in range(D // 16):
                zbuf_v[r, pl.ds(g * 16, 16)] = zero
            return carry

        lax.fori_loop(0, ZCHUNK, zrow, 0)
        for z in range(ROWS_PER_TILE // ZCHUNK):
            ro = sid * ROWS_PER_TILE + z * ZCHUNK
            pltpu.sync_copy(zbuf_v, acc_sh.at[pl.ds(ro, ZCHUNK)])
        plsc.subcore_barrier()

        def ebase(c):
            return pl.multiple_of(wid * e_per_w + c * CHUNK, 8)

        def issue_idx(c, q):
            pltpu.async_copy(src_hbm.at[pl.ds(ebase(c), CHUNK)], srcb[q],
                             isems[q])
            pltpu.async_copy(dst_hbm.at[pl.ds(ebase(c), CHUNK)], dstb[q],
                             isems[q])

        def wait_idx(c, q):
            pltpu.make_async_copy(src_hbm.at[pl.ds(ebase(c), CHUNK)],
                                  srcb[q], isems[q]).wait()
            pltpu.make_async_copy(dst_hbm.at[pl.ds(ebase(c), CHUNK)],
                                  dstb[q], isems[q]).wait()

        def issue_eproj(c, q):
            pltpu.async_copy(eproj_hbm.at[pl.ds(ebase(c), CHUNK)], erb[q],
                             epsems[q])

        def issue_ga(c, q):
            pltpu.make_async_copy(eproj_hbm.at[pl.ds(ebase(c), CHUNK)],
                                  erb[q], epsems[q]).wait()
            wait_idx(c, q)
            pltpu.async_copy(x_hbm.at[srcb[q]], erb[q], gasems[q], add=True)

        def process(c, p, traced):
            def when(cond):
                if traced:
                    return pl.when(cond)
                return _run_if(bool(cond))

            @when(c + 2 < n_chunks)
            def _():
                issue_idx(c + 2, (p + 2) % 4)
                issue_eproj(c + 2, (p + 2) % 4)

            @when(c + 1 < n_chunks)
            def _():
                issue_ga(c + 1, (p + 1) % 4)

            pltpu.make_async_copy(x_hbm.at[srcb[p]], erb[p],
                                  gasems[p]).wait()
            er = erb[p]

            @plsc.parallel_loop(0, CHUNK, step=1, unroll=4)
            def row_body(r):
                for g in range(D // 16):
                    sl = pl.ds(g * 16, 16)
                    er[r, sl] = jnp.maximum(er[r, sl], 0.0)

            @when(c >= 1)
            def _():
                pq = (p + 3) % 4
                pltpu.make_async_copy(erb[pq], acc_sh.at[dstb[pq]],
                                      scsem).wait()

            pltpu.async_copy(er, acc_sh.at[dstb[p]], scsem, add=True)

        issue_idx(0, 0)
        issue_idx(1, 1)
        issue_eproj(0, 0)
        issue_eproj(1, 1)
        issue_ga(0, 0)

        def quad_body(k, carry):
            for b4 in range(4):
                process(4 * k + b4, b4, traced=True)
            return carry

        lax.fori_loop(0, n_chunks // 4, quad_body, 0)
        for c in range(4 * (n_chunks // 4), n_chunks):
            process(c, c % 4, traced=False)
        lastq = (n_chunks - 1) % 4
        pltpu.make_async_copy(erb[lastq], acc_sh.at[dstb[lastq]],
                              scsem).wait()
        plsc.subcore_barrier()

        for z in range(ROWS_PER_TILE // ZCHUNK):
            ro = sid * ROWS_PER_TILE + z * ZCHUNK
            pltpu.sync_copy(acc_sh.at[pl.ds(ro, ZCHUNK)], zbuf_v)
            pltpu.sync_copy(
                zbuf_v, out_hbm.at[pl.ds(cid * N_PAD + ro, ZCHUNK)])

    return _sc_aggregate


_sc_slab = _make_sc_aggregate(E_SLAB // NW)


def kernel(x, edge_index, edge_attr, lin_W, lin_b, nn_W, nn_b):
    src = edge_index[0].astype(jnp.int32)
    dst = edge_index[1].astype(jnp.int32)
    lin_Wt = lin_W.T
    parts = []
    for s in range(SLABS):
        lo, hi = s * E_SLAB, (s + 1) * E_SLAB
        e_proj = _edge_proj(edge_attr[lo:hi], lin_Wt, lin_b)
        p = _sc_slab(src[lo:hi], dst[lo:hi], x, e_proj)
        parts.append(p[:N_NODES])
        parts.append(p[N_PAD:N_PAD + N_NODES])
    return _combine(x, parts, nn_W.T, nn_b)


# speedup vs baseline: 1.3677x; 1.1317x over previous
import functools

import jax
import jax.numpy as jnp
from jax import lax
from jax.experimental import pallas as pl
from jax.experimental.pallas import tpu as pltpu
from jax.experimental.pallas import tpu_sc as plsc

N_NODES = 10000
N_EDGES = 320000
D = 128

NC = 2
NS = 16
NW = NC * NS
SLABS = 1
E_SLAB = N_EDGES // SLABS
CHUNK = 40
N_PAD = 10240
ROWS_PER_TILE = N_PAD // NS
ZCHUNK = 64

BE = 8000
BN = 2000


def _mm_bias_body(a_ref, w_ref, b_ref, o_ref):
    o_ref[...] = (
        jnp.dot(a_ref[...].astype(jnp.bfloat16), w_ref[...],
                preferred_element_type=jnp.float32)
        + b_ref[...]
    )


def _edge_proj(edge_attr, lin_Wt, lin_b):
    e = edge_attr.shape[0]
    return pl.pallas_call(
        _mm_bias_body,
        grid=(e // BE,),
        in_specs=[
            pl.BlockSpec((BE, D), lambda i: (i, 0)),
            pl.BlockSpec((D, D), lambda i: (0, 0)),
            pl.BlockSpec((1, D), lambda i: (0, 0)),
        ],
        out_specs=pl.BlockSpec((BE, D), lambda i: (i, 0)),
        out_shape=jax.ShapeDtypeStruct((e, D), jnp.float32),
    )(edge_attr, lin_Wt.astype(jnp.bfloat16), lin_b.reshape(1, D))


def _combine(x, parts, nn_Wt, nn_b):
    n = len(parts)

    def body(*refs):
        x_ref, prefs, (w_ref, b_ref, o_ref) = refs[0], refs[1:1 + n], refs[1 + n:]
        h = x_ref[...]
        for p_ref in prefs:
            h = h + p_ref[...]
        o_ref[...] = (
            jnp.dot(h, w_ref[...], preferred_element_type=jnp.float32)
            + b_ref[...]
        )

    node_spec = pl.BlockSpec((BN, D), lambda i: (i, 0))
    return pl.pallas_call(
        body,
        grid=(N_NODES // BN,),
        in_specs=[node_spec] * (1 + n) + [
            pl.BlockSpec((D, D), lambda i: (0, 0)),
            pl.BlockSpec((1, D), lambda i: (0, 0)),
        ],
        out_specs=node_spec,
        out_shape=jax.ShapeDtypeStruct((N_NODES, D), jnp.float32),
    )(x, *parts, nn_Wt, nn_b.reshape(1, D))


def _run_if(cond):
    def deco(f):
        if cond:
            f()
    return deco


_sc_mesh = plsc.VectorSubcoreMesh(core_axis_name="c", subcore_axis_name="s")


def _make_sc_aggregate(e_per_w):
    n_chunks = e_per_w // CHUNK

    @functools.partial(
        pl.kernel,
        out_type=jax.ShapeDtypeStruct((NC * N_PAD, D), jnp.float32),
        mesh=_sc_mesh,
        scratch_types=[pltpu.VMEM((CHUNK,), jnp.int32)] * 8
        + [pltpu.VMEM((CHUNK, D), jnp.float32)] * 4
        + [
            pltpu.VMEM((ZCHUNK, D), jnp.float32),
            pltpu.VMEM_SHARED((N_PAD, D), jnp.float32),
        ] + [pltpu.SemaphoreType.DMA] * 13,
    )
    def _sc_aggregate(src_hbm, dst_hbm, x_hbm, eproj_hbm, out_hbm,
                      s0, s1, s2, s3, d0, d1, d2, d3,
                      er0, er1, er2, er3, zbuf_v, acc_sh,
                      *sems):
        cid = lax.axis_index("c")
        sid = lax.axis_index("s")
        wid = sid * NC + cid
        srcb = (s0, s1, s2, s3)
        dstb = (d0, d1, d2, d3)
        erb = (er0, er1, er2, er3)
        isems = sems[0:4]
        epsems = sems[4:8]
        gasems = sems[8:12]
        scsem = sems[12]

        zero = jnp.zeros((16,), jnp.float32)

        def zrow(r, carry):
            for g in range(D // 16):
                zbuf_v[r, pl.ds(g * 16, 16)] = zero
            return carry

        lax.fori_loop(0, ZCHUNK, zrow, 0)
        for z in range(ROWS_PER_TILE // ZCHUNK):
            ro = sid * ROWS_PER_TILE + z * ZCHUNK
            pltpu.sync_copy(zbuf_v, acc_sh.at[pl.ds(ro, ZCHUNK)])
        plsc.subcore_barrier()

        def ebase(c):
            return pl.multiple_of(wid * e_per_w + c * CHUNK, 8)

        def issue_idx(c, q):
            pltpu.async_copy(src_hbm.at[pl.ds(ebase(c), CHUNK)], srcb[q],
                             isems[q])
            pltpu.async_copy(dst_hbm.at[pl.ds(ebase(c), CHUNK)], dstb[q],
                             isems[q])

        def wait_idx(c, q):
            pltpu.make_async_copy(src_hbm.at[pl.ds(ebase(c), CHUNK)],
                                  srcb[q], isems[q]).wait()
            pltpu.make_async_copy(dst_hbm.at[pl.ds(ebase(c), CHUNK)],
                                  dstb[q], isems[q]).wait()

        def issue_eproj(c, q):
            pltpu.async_copy(eproj_hbm.at[pl.ds(ebase(c), CHUNK)], erb[q],
                             epsems[q])

        def issue_ga(c, q):
            pltpu.make_async_copy(eproj_hbm.at[pl.ds(ebase(c), CHUNK)],
                                  erb[q], epsems[q]).wait()
            wait_idx(c, q)
            pltpu.async_copy(x_hbm.at[srcb[q]], erb[q], gasems[q], add=True)

        def process(c, p, traced):
            def when(cond):
                if traced:
                    return pl.when(cond)
                return _run_if(bool(cond))

            @when(c + 2 < n_chunks)
            def _():
                issue_idx(c + 2, (p + 2) % 4)
                issue_eproj(c + 2, (p + 2) % 4)

            @when(c + 1 < n_chunks)
            def _():
                issue_ga(c + 1, (p + 1) % 4)

            pltpu.make_async_copy(x_hbm.at[srcb[p]], erb[p],
                                  gasems[p]).wait()
            er = erb[p]

            @plsc.parallel_loop(0, CHUNK, step=1, unroll=4)
            def row_body(r):
                for g in range(D // 16):
                    sl = pl.ds(g * 16, 16)
                    er[r, sl] = jnp.maximum(er[r, sl], 0.0)

            @when(c >= 1)
            def _():
                pq = (p + 3) % 4
                pltpu.make_async_copy(erb[pq], acc_sh.at[dstb[pq]],
                                      scsem).wait()

            pltpu.async_copy(er, acc_sh.at[dstb[p]], scsem, add=True)

        issue_idx(0, 0)
        issue_idx(1, 1)
        issue_eproj(0, 0)
        issue_eproj(1, 1)
        issue_ga(0, 0)

        def quad_body(k, carry):
            for b4 in range(4):
                process(4 * k + b4, b4, traced=True)
            return carry

        lax.fori_loop(0, n_chunks // 4, quad_body, 0)
        for c in range(4 * (n_chunks // 4), n_chunks):
            process(c, c % 4, traced=False)
        lastq = (n_chunks - 1) % 4
        pltpu.make_async_copy(erb[lastq], acc_sh.at[dstb[lastq]],
                              scsem).wait()
        plsc.subcore_barrier()

        for z in range(ROWS_PER_TILE // ZCHUNK):
            ro = sid * ROWS_PER_TILE + z * ZCHUNK
            pltpu.sync_copy(acc_sh.at[pl.ds(ro, ZCHUNK)], zbuf_v)
            pltpu.sync_copy(
                zbuf_v, out_hbm.at[pl.ds(cid * N_PAD + ro, ZCHUNK)])

    return _sc_aggregate


_sc_slab = _make_sc_aggregate(E_SLAB // NW)


def kernel(x, edge_index, edge_attr, lin_W, lin_b, nn_W, nn_b):
    src = edge_index[0].astype(jnp.int32)
    dst = edge_index[1].astype(jnp.int32)
    lin_Wt = lin_W.T
    parts = []
    for s in range(SLABS):
        lo, hi = s * E_SLAB, (s + 1) * E_SLAB
        e_proj = _edge_proj(edge_attr[lo:hi], lin_Wt, lin_b)
        p = _sc_slab(src[lo:hi], dst[lo:hi], x, e_proj)
        parts.append(p[:N_NODES])
        parts.append(p[N_PAD:N_PAD + N_NODES])
    return _combine(x, parts, nn_W.T, nn_b)


# speedup vs baseline: 1.3757x; 1.0059x over previous
import functools

import jax
import jax.numpy as jnp
from jax import lax
from jax.experimental import pallas as pl
from jax.experimental.pallas import tpu as pltpu
from jax.experimental.pallas import tpu_sc as plsc

N_NODES = 10000
N_EDGES = 320000
D = 128

NC = 2
NS = 16
NW = NC * NS
SLABS = 1
E_SLAB = N_EDGES // SLABS
CHUNK = 40
N_PAD = 10240
ROWS_PER_TILE = N_PAD // NS
ZCHUNK = 64

BE = 16000
BN = 2000


def _mm_bias_body(a_ref, w_ref, b_ref, o_ref):
    o_ref[...] = (
        jnp.dot(a_ref[...].astype(jnp.bfloat16), w_ref[...],
                preferred_element_type=jnp.float32)
        + b_ref[...]
    )


def _edge_proj(edge_attr, lin_Wt, lin_b):
    e = edge_attr.shape[0]
    return pl.pallas_call(
        _mm_bias_body,
        grid=(e // BE,),
        in_specs=[
            pl.BlockSpec((BE, D), lambda i: (i, 0)),
            pl.BlockSpec((D, D), lambda i: (0, 0)),
            pl.BlockSpec((1, D), lambda i: (0, 0)),
        ],
        out_specs=pl.BlockSpec((BE, D), lambda i: (i, 0)),
        out_shape=jax.ShapeDtypeStruct((e, D), jnp.float32),
    )(edge_attr, lin_Wt.astype(jnp.bfloat16), lin_b.reshape(1, D))


def _combine(x, parts, nn_Wt, nn_b):
    n = len(parts)

    def body(*refs):
        x_ref, prefs, (w_ref, b_ref, o_ref) = refs[0], refs[1:1 + n], refs[1 + n:]
        h = x_ref[...]
        for p_ref in prefs:
            h = h + p_ref[...]
        o_ref[...] = (
            jnp.dot(h, w_ref[...], preferred_element_type=jnp.float32)
            + b_ref[...]
        )

    node_spec = pl.BlockSpec((BN, D), lambda i: (i, 0))
    return pl.pallas_call(
        body,
        grid=(N_NODES // BN,),
        in_specs=[node_spec] * (1 + n) + [
            pl.BlockSpec((D, D), lambda i: (0, 0)),
            pl.BlockSpec((1, D), lambda i: (0, 0)),
        ],
        out_specs=node_spec,
        out_shape=jax.ShapeDtypeStruct((N_NODES, D), jnp.float32),
    )(x, *parts, nn_Wt, nn_b.reshape(1, D))


def _run_if(cond):
    def deco(f):
        if cond:
            f()
    return deco


_sc_mesh = plsc.VectorSubcoreMesh(core_axis_name="c", subcore_axis_name="s")


def _make_sc_aggregate(e_per_w):
    n_chunks = e_per_w // CHUNK

    @functools.partial(
        pl.kernel,
        out_type=jax.ShapeDtypeStruct((NC * N_PAD, D), jnp.float32),
        mesh=_sc_mesh,
        scratch_types=[pltpu.VMEM((CHUNK,), jnp.int32)] * 8
        + [pltpu.VMEM((CHUNK, D), jnp.float32)] * 4
        + [
            pltpu.VMEM((ZCHUNK, D), jnp.float32),
            pltpu.VMEM_SHARED((N_PAD, D), jnp.float32),
        ] + [pltpu.SemaphoreType.DMA] * 13,
    )
    def _sc_aggregate(src_hbm, dst_hbm, x_hbm, eproj_hbm, out_hbm,
                      s0, s1, s2, s3, d0, d1, d2, d3,
                      er0, er1, er2, er3, zbuf_v, acc_sh,
                      *sems):
        cid = lax.axis_index("c")
        sid = lax.axis_index("s")
        wid = sid * NC + cid
        srcb = (s0, s1, s2, s3)
        dstb = (d0, d1, d2, d3)
        erb = (er0, er1, er2, er3)
        isems = sems[0:4]
        epsems = sems[4:8]
        gasems = sems[8:12]
        scsem = sems[12]

        zero = jnp.zeros((16,), jnp.float32)

        def zrow(r, carry):
            for g in range(D // 16):
                zbuf_v[r, pl.ds(g * 16, 16)] = zero
            return carry

        lax.fori_loop(0, ZCHUNK, zrow, 0)
        for z in range(ROWS_PER_TILE // ZCHUNK):
            ro = sid * ROWS_PER_TILE + z * ZCHUNK
            pltpu.sync_copy(zbuf_v, acc_sh.at[pl.ds(ro, ZCHUNK)])
        plsc.subcore_barrier()

        def ebase(c):
            return pl.multiple_of(wid * e_per_w + c * CHUNK, 8)

        def issue_idx(c, q):
            pltpu.async_copy(src_hbm.at[pl.ds(ebase(c), CHUNK)], srcb[q],
                             isems[q])
            pltpu.async_copy(dst_hbm.at[pl.ds(ebase(c), CHUNK)], dstb[q],
                             isems[q])

        def wait_idx(c, q):
            pltpu.make_async_copy(src_hbm.at[pl.ds(ebase(c), CHUNK)],
                                  srcb[q], isems[q]).wait()
            pltpu.make_async_copy(dst_hbm.at[pl.ds(ebase(c), CHUNK)],
                                  dstb[q], isems[q]).wait()

        def issue_eproj(c, q):
            pltpu.async_copy(eproj_hbm.at[pl.ds(ebase(c), CHUNK)], erb[q],
                             epsems[q])

        def issue_ga(c, q):
            pltpu.make_async_copy(eproj_hbm.at[pl.ds(ebase(c), CHUNK)],
                                  erb[q], epsems[q]).wait()
            wait_idx(c, q)
            pltpu.async_copy(x_hbm.at[srcb[q]], erb[q], gasems[q], add=True)

        def process(c, p, traced):
            def when(cond):
                if traced:
                    return pl.when(cond)
                return _run_if(bool(cond))

            @when(c + 2 < n_chunks)
            def _():
                issue_idx(c + 2, (p + 2) % 4)
                issue_eproj(c + 2, (p + 2) % 4)

            @when(c + 1 < n_chunks)
            def _():
                issue_ga(c + 1, (p + 1) % 4)

            pltpu.make_async_copy(x_hbm.at[srcb[p]], erb[p],
                                  gasems[p]).wait()
            er = erb[p]

            @plsc.parallel_loop(0, CHUNK, step=1, unroll=4)
            def row_body(r):
                for g in range(D // 16):
                    sl = pl.ds(g * 16, 16)
                    er[r, sl] = jnp.maximum(er[r, sl], 0.0)

            @when(c >= 1)
            def _():
                pq = (p + 3) % 4
                pltpu.make_async_copy(erb[pq], acc_sh.at[dstb[pq]],
                                      scsem).wait()

            pltpu.async_copy(er, acc_sh.at[dstb[p]], scsem, add=True)

        issue_idx(0, 0)
        issue_idx(1, 1)
        issue_eproj(0, 0)
        issue_eproj(1, 1)
        issue_ga(0, 0)

        def quad_body(k, carry):
            for b4 in range(4):
                process(4 * k + b4, b4, traced=True)
            return carry

        lax.fori_loop(0, n_chunks // 4, quad_body, 0)
        for c in range(4 * (n_chunks // 4), n_chunks):
            process(c, c % 4, traced=False)
        lastq = (n_chunks - 1) % 4
        pltpu.make_async_copy(erb[lastq], acc_sh.at[dstb[lastq]],
                              scsem).wait()
        plsc.subcore_barrier()

        for z in range(ROWS_PER_TILE // ZCHUNK):
            ro = sid * ROWS_PER_TILE + z * ZCHUNK
            pltpu.sync_copy(acc_sh.at[pl.ds(ro, ZCHUNK)], zbuf_v)
            pltpu.sync_copy(
                zbuf_v, out_hbm.at[pl.ds(cid * N_PAD + ro, ZCHUNK)])

    return _sc_aggregate


_sc_slab = _make_sc_aggregate(E_SLAB // NW)


def kernel(x, edge_index, edge_attr, lin_W, lin_b, nn_W, nn_b):
    src = edge_index[0].astype(jnp.int32)
    dst = edge_index[1].astype(jnp.int32)
    lin_Wt = lin_W.T
    parts = []
    for s in range(SLABS):
        lo, hi = s * E_SLAB, (s + 1) * E_SLAB
        e_proj = _edge_proj(edge_attr[lo:hi], lin_Wt, lin_b)
        p = _sc_slab(src[lo:hi], dst[lo:hi], x, e_proj)
        parts.append(p[:N_NODES])
        parts.append(p[N_PAD:N_PAD + N_NODES])
    return _combine(x, parts, nn_W.T, nn_b)


# speedup vs baseline: 1.4039x; 1.0205x over previous
import functools

import jax
import jax.numpy as jnp
from jax import lax
from jax.experimental import pallas as pl
from jax.experimental.pallas import tpu as pltpu
from jax.experimental.pallas import tpu_sc as plsc

N_NODES = 10000
N_EDGES = 320000
D = 128

NC = 2
NS = 16
NW = NC * NS
SLABS = 1
E_SLAB = N_EDGES // SLABS
CHUNK = 40
N_PAD = 10240
ROWS_PER_TILE = N_PAD // NS
ZCHUNK = 64

BE = 16000
BN = 2000


def _mm_bias_body(a_ref, w_ref, b_ref, o_ref):
    o_ref[...] = (
        jnp.dot(a_ref[...].astype(jnp.bfloat16), w_ref[...],
                preferred_element_type=jnp.float32)
        + b_ref[...]
    )


def _edge_proj(edge_attr, lin_Wt, lin_b):
    e = edge_attr.shape[0]
    return pl.pallas_call(
        _mm_bias_body,
        grid=(e // BE,),
        in_specs=[
            pl.BlockSpec((BE, D), lambda i: (i, 0)),
            pl.BlockSpec((D, D), lambda i: (0, 0)),
            pl.BlockSpec((1, D), lambda i: (0, 0)),
        ],
        out_specs=pl.BlockSpec((BE, D), lambda i: (i, 0)),
        out_shape=jax.ShapeDtypeStruct((e, D), jnp.float32),
    )(edge_attr, lin_Wt.astype(jnp.bfloat16), lin_b.reshape(1, D))


def _combine(x, parts, nn_Wt, nn_b):
    n = len(parts)

    def body(*refs):
        x_ref, prefs, (w_ref, b_ref, o_ref) = (
            refs[0], refs[1:1 + 2 * n], refs[1 + 2 * n:])
        h = x_ref[...]
        for p_ref in prefs:
            h = h + p_ref[0]
        o_ref[...] = (
            jnp.dot(h, w_ref[...], preferred_element_type=jnp.float32)
            + b_ref[...]
        )

    node_spec = pl.BlockSpec((BN, D), lambda i: (i, 0))
    part_specs = []
    for _ in range(n):
        part_specs.append(pl.BlockSpec((1, BN, D), lambda i: (0, i, 0)))
        part_specs.append(pl.BlockSpec((1, BN, D), lambda i: (1, i, 0)))
    part_args = []
    for p in parts:
        part_args.extend([p, p])
    return pl.pallas_call(
        body,
        grid=(N_NODES // BN,),
        in_specs=[node_spec] + part_specs + [
            pl.BlockSpec((D, D), lambda i: (0, 0)),
            pl.BlockSpec((1, D), lambda i: (0, 0)),
        ],
        out_specs=node_spec,
        out_shape=jax.ShapeDtypeStruct((N_NODES, D), jnp.float32),
    )(x, *part_args, nn_Wt, nn_b.reshape(1, D))


def _run_if(cond):
    def deco(f):
        if cond:
            f()
    return deco


_sc_mesh = plsc.VectorSubcoreMesh(core_axis_name="c", subcore_axis_name="s")


def _make_sc_aggregate(e_per_w):
    n_chunks = e_per_w // CHUNK

    @functools.partial(
        pl.kernel,
        out_type=jax.ShapeDtypeStruct((NC * N_PAD, D), jnp.float32),
        mesh=_sc_mesh,
        scratch_types=[pltpu.VMEM((CHUNK,), jnp.int32)] * 8
        + [pltpu.VMEM((CHUNK, D), jnp.float32)] * 4
        + [
            pltpu.VMEM((ZCHUNK, D), jnp.float32),
            pltpu.VMEM_SHARED((N_PAD, D), jnp.float32),
        ] + [pltpu.SemaphoreType.DMA] * 13,
    )
    def _sc_aggregate(src_hbm, dst_hbm, x_hbm, eproj_hbm, out_hbm,
                      s0, s1, s2, s3, d0, d1, d2, d3,
                      er0, er1, er2, er3, zbuf_v, acc_sh,
                      *sems):
        cid = lax.axis_index("c")
        sid = lax.axis_index("s")
        wid = sid * NC + cid
        srcb = (s0, s1, s2, s3)
        dstb = (d0, d1, d2, d3)
        erb = (er0, er1, er2, er3)
        isems = sems[0:4]
        epsems = sems[4:8]
        gasems = sems[8:12]
        scsem = sems[12]

        zero = jnp.zeros((16,), jnp.float32)

        def zrow(r, carry):
            for g in range(D // 16):
                zbuf_v[r, pl.ds(g * 16, 16)] = zero
            return carry

        lax.fori_loop(0, ZCHUNK, zrow, 0)
        for z in range(ROWS_PER_TILE // ZCHUNK):
            ro = sid * ROWS_PER_TILE + z * ZCHUNK
            pltpu.sync_copy(zbuf_v, acc_sh.at[pl.ds(ro, ZCHUNK)])
        plsc.subcore_barrier()

        def ebase(c):
            return pl.multiple_of(wid * e_per_w + c * CHUNK, 8)

        def issue_idx(c, q):
            pltpu.async_copy(src_hbm.at[pl.ds(ebase(c), CHUNK)], srcb[q],
                             isems[q])
            pltpu.async_copy(dst_hbm.at[pl.ds(ebase(c), CHUNK)], dstb[q],
                             isems[q])

        def wait_idx(c, q):
            pltpu.make_async_copy(src_hbm.at[pl.ds(ebase(c), CHUNK)],
                                  srcb[q], isems[q]).wait()
            pltpu.make_async_copy(dst_hbm.at[pl.ds(ebase(c), CHUNK)],
                                  dstb[q], isems[q]).wait()

        def issue_eproj(c, q):
            pltpu.async_copy(eproj_hbm.at[pl.ds(ebase(c), CHUNK)], erb[q],
                             epsems[q])

        def issue_ga(c, q):
            pltpu.make_async_copy(eproj_hbm.at[pl.ds(ebase(c), CHUNK)],
                                  erb[q], epsems[q]).wait()
            wait_idx(c, q)
            pltpu.async_copy(x_hbm.at[srcb[q]], erb[q], gasems[q], add=True)

        def process(c, p, traced):
            def when(cond):
                if traced:
                    return pl.when(cond)
                return _run_if(bool(cond))

            @when(c + 2 < n_chunks)
            def _():
                issue_idx(c + 2, (p + 2) % 4)
                issue_eproj(c + 2, (p + 2) % 4)

            @when(c + 1 < n_chunks)
            def _():
                issue_ga(c + 1, (p + 1) % 4)

            pltpu.make_async_copy(x_hbm.at[srcb[p]], erb[p],
                                  gasems[p]).wait()
            er = erb[p]

            @plsc.parallel_loop(0, CHUNK, step=1, unroll=4)
            def row_body(r):
                for g in range(D // 16):
                    sl = pl.ds(g * 16, 16)
                    er[r, sl] = jnp.maximum(er[r, sl], 0.0)

            @when(c >= 1)
            def _():
                pq = (p + 3) % 4
                pltpu.make_async_copy(erb[pq], acc_sh.at[dstb[pq]],
                                      scsem).wait()

            pltpu.async_copy(er, acc_sh.at[dstb[p]], scsem, add=True)

        issue_idx(0, 0)
        issue_idx(1, 1)
        issue_eproj(0, 0)
        issue_eproj(1, 1)
        issue_ga(0, 0)

        def quad_body(k, carry):
            for b4 in range(4):
                process(4 * k + b4, b4, traced=True)
            return carry

        lax.fori_loop(0, n_chunks // 4, quad_body, 0)
        for c in range(4 * (n_chunks // 4), n_chunks):
            process(c, c % 4, traced=False)
        lastq = (n_chunks - 1) % 4
        pltpu.make_async_copy(erb[lastq], acc_sh.at[dstb[lastq]],
                              scsem).wait()
        plsc.subcore_barrier()

        for z in range(ROWS_PER_TILE // ZCHUNK):
            ro = sid * ROWS_PER_TILE + z * ZCHUNK
            pltpu.sync_copy(acc_sh.at[pl.ds(ro, ZCHUNK)], zbuf_v)
            pltpu.sync_copy(
                zbuf_v, out_hbm.at[pl.ds(cid * N_PAD + ro, ZCHUNK)])

    return _sc_aggregate


_sc_slab = _make_sc_aggregate(E_SLAB // NW)


def kernel(x, edge_index, edge_attr, lin_W, lin_b, nn_W, nn_b):
    src = edge_index[0].astype(jnp.int32)
    dst = edge_index[1].astype(jnp.int32)
    e_proj = _edge_proj(edge_attr, lin_W.T, lin_b)
    p = _sc_slab(src, dst, x, e_proj)
    return _combine(x, [p.reshape(NC, N_PAD, D)], nn_W.T, nn_b)


# speedup vs baseline: 1.4071x; 1.0022x over previous
import functools

import jax
import jax.numpy as jnp
from jax import lax
from jax.experimental import pallas as pl
from jax.experimental.pallas import tpu as pltpu
from jax.experimental.pallas import tpu_sc as plsc

N_NODES = 10000
N_EDGES = 320000
D = 128

NC = 2
NS = 16
NW = NC * NS
SLABS = 1
E_SLAB = N_EDGES // SLABS
CHUNK = 40
N_PAD = 10240
ROWS_PER_TILE = N_PAD // NS
ZCHUNK = 64

BE = 16000
BN = 5000


def _mm_bias_body(a_ref, w_ref, b_ref, o_ref):
    o_ref[...] = (
        jnp.dot(a_ref[...].astype(jnp.bfloat16), w_ref[...],
                preferred_element_type=jnp.float32)
        + b_ref[...]
    )


def _edge_proj(edge_attr, lin_Wt, lin_b):
    e = edge_attr.shape[0]
    return pl.pallas_call(
        _mm_bias_body,
        grid=(e // BE,),
        in_specs=[
            pl.BlockSpec((BE, D), lambda i: (i, 0)),
            pl.BlockSpec((D, D), lambda i: (0, 0)),
            pl.BlockSpec((1, D), lambda i: (0, 0)),
        ],
        out_specs=pl.BlockSpec((BE, D), lambda i: (i, 0)),
        out_shape=jax.ShapeDtypeStruct((e, D), jnp.float32),
    )(edge_attr, lin_Wt.astype(jnp.bfloat16), lin_b.reshape(1, D))


def _combine(x, parts, nn_Wt, nn_b):
    n = len(parts)

    def body(*refs):
        x_ref, prefs, (w_ref, b_ref, o_ref) = (
            refs[0], refs[1:1 + 2 * n], refs[1 + 2 * n:])
        h = x_ref[...]
        for p_ref in prefs:
            h = h + p_ref[0]
        o_ref[...] = (
            jnp.dot(h, w_ref[...], preferred_element_type=jnp.float32)
            + b_ref[...]
        )

    node_spec = pl.BlockSpec((BN, D), lambda i: (i, 0))
    part_specs = []
    for _ in range(n):
        part_specs.append(pl.BlockSpec((1, BN, D), lambda i: (0, i, 0)))
        part_specs.append(pl.BlockSpec((1, BN, D), lambda i: (1, i, 0)))
    part_args = []
    for p in parts:
        part_args.extend([p, p])
    return pl.pallas_call(
        body,
        grid=(N_NODES // BN,),
        in_specs=[node_spec] + part_specs + [
            pl.BlockSpec((D, D), lambda i: (0, 0)),
            pl.BlockSpec((1, D), lambda i: (0, 0)),
        ],
        out_specs=node_spec,
        out_shape=jax.ShapeDtypeStruct((N_NODES, D), jnp.float32),
    )(x, *part_args, nn_Wt, nn_b.reshape(1, D))


def _run_if(cond):
    def deco(f):
        if cond:
            f()
    return deco


_sc_mesh = plsc.VectorSubcoreMesh(core_axis_name="c", subcore_axis_name="s")


def _make_sc_aggregate(e_per_w):
    n_chunks = e_per_w // CHUNK

    @functools.partial(
        pl.kernel,
        out_type=jax.ShapeDtypeStruct((NC * N_PAD, D), jnp.float32),
        mesh=_sc_mesh,
        scratch_types=[pltpu.VMEM((CHUNK,), jnp.int32)] * 8
        + [pltpu.VMEM((CHUNK, D), jnp.float32)] * 4
        + [
            pltpu.VMEM((ZCHUNK, D), jnp.float32),
            pltpu.VMEM_SHARED((N_PAD, D), jnp.float32),
        ] + [pltpu.SemaphoreType.DMA] * 13,
    )
    def _sc_aggregate(src_hbm, dst_hbm, x_hbm, eproj_hbm, out_hbm,
                      s0, s1, s2, s3, d0, d1, d2, d3,
                      er0, er1, er2, er3, zbuf_v, acc_sh,
                      *sems):
        cid = lax.axis_index("c")
        sid = lax.axis_index("s")
        wid = sid * NC + cid
        srcb = (s0, s1, s2, s3)
        dstb = (d0, d1, d2, d3)
        erb = (er0, er1, er2, er3)
        isems = sems[0:4]
        epsems = sems[4:8]
        gasems = sems[8:12]
        scsem = sems[12]

        zero = jnp.zeros((16,), jnp.float32)

        def zrow(r, carry):
            for g in range(D // 16):
                zbuf_v[r, pl.ds(g * 16, 16)] = zero
            return carry

        lax.fori_loop(0, ZCHUNK, zrow, 0)
        for z in range(ROWS_PER_TILE // ZCHUNK):
            ro = sid * ROWS_PER_TILE + z * ZCHUNK
            pltpu.sync_copy(zbuf_v, acc_sh.at[pl.ds(ro, ZCHUNK)])
        plsc.subcore_barrier()

        def ebase(c):
            return pl.multiple_of(wid * e_per_w + c * CHUNK, 8)

        def issue_idx(c, q):
            pltpu.async_copy(src_hbm.at[pl.ds(ebase(c), CHUNK)], srcb[q],
                             isems[q])
            pltpu.async_copy(dst_hbm.at[pl.ds(ebase(c), CHUNK)], dstb[q],
                             isems[q])

        def wait_idx(c, q):
            pltpu.make_async_copy(src_hbm.at[pl.ds(ebase(c), CHUNK)],
                                  srcb[q], isems[q]).wait()
            pltpu.make_async_copy(dst_hbm.at[pl.ds(ebase(c), CHUNK)],
                                  dstb[q], isems[q]).wait()

        def issue_eproj(c, q):
            pltpu.async_copy(eproj_hbm.at[pl.ds(ebase(c), CHUNK)], erb[q],
                             epsems[q])

        def issue_ga(c, q):
            pltpu.make_async_copy(eproj_hbm.at[pl.ds(ebase(c), CHUNK)],
                                  erb[q], epsems[q]).wait()
            wait_idx(c, q)
            pltpu.async_copy(x_hbm.at[srcb[q]], erb[q], gasems[q], add=True)

        def process(c, p, traced):
            def when(cond):
                if traced:
                    return pl.when(cond)
                return _run_if(bool(cond))

            @when(c + 2 < n_chunks)
            def _():
                issue_idx(c + 2, (p + 2) % 4)
                issue_eproj(c + 2, (p + 2) % 4)

            @when(c + 1 < n_chunks)
            def _():
                issue_ga(c + 1, (p + 1) % 4)

            pltpu.make_async_copy(x_hbm.at[srcb[p]], erb[p],
                                  gasems[p]).wait()
            er = erb[p]

            @plsc.parallel_loop(0, CHUNK, step=1, unroll=8)
            def row_body(r):
                for g in range(D // 16):
                    sl = pl.ds(g * 16, 16)
                    er[r, sl] = jnp.maximum(er[r, sl], 0.0)

            @when(c >= 1)
            def _():
                pq = (p + 3) % 4
                pltpu.make_async_copy(erb[pq], acc_sh.at[dstb[pq]],
                                      scsem).wait()

            pltpu.async_copy(er, acc_sh.at[dstb[p]], scsem, add=True)

        issue_idx(0, 0)
        issue_idx(1, 1)
        issue_eproj(0, 0)
        issue_eproj(1, 1)
        issue_ga(0, 0)

        def quad_body(k, carry):
            for b4 in range(4):
                process(4 * k + b4, b4, traced=True)
            return carry

        lax.fori_loop(0, n_chunks // 4, quad_body, 0)
        for c in range(4 * (n_chunks // 4), n_chunks):
            process(c, c % 4, traced=False)
        lastq = (n_chunks - 1) % 4
        pltpu.make_async_copy(erb[lastq], acc_sh.at[dstb[lastq]],
                              scsem).wait()
        plsc.subcore_barrier()

        for z in range(ROWS_PER_TILE // ZCHUNK):
            ro = sid * ROWS_PER_TILE + z * ZCHUNK
            pltpu.sync_copy(acc_sh.at[pl.ds(ro, ZCHUNK)], zbuf_v)
            pltpu.sync_copy(
                zbuf_v, out_hbm.at[pl.ds(cid * N_PAD + ro, ZCHUNK)])

    return _sc_aggregate


_sc_slab = _make_sc_aggregate(E_SLAB // NW)


def kernel(x, edge_index, edge_attr, lin_W, lin_b, nn_W, nn_b):
    src = edge_index[0].astype(jnp.int32)
    dst = edge_index[1].astype(jnp.int32)
    e_proj = _edge_proj(edge_attr, lin_W.T, lin_b)
    p = _sc_slab(src, dst, x, e_proj)
    return _combine(x, [p.reshape(NC, N_PAD, D)], nn_W.T, nn_b)
